# baseline (speedup 1.0000x reference)
import functools
import jax
import jax.numpy as jnp
from jax.experimental import pallas as pl
from jax.experimental.pallas import tpu as pltpu

P = 3
L = 8
_DIRS = [(0, 1), (1, 0), (1, 1), (1, -1)]
_PAIRS = []
for _dr, _dc in _DIRS:
    for _i in range(P):
        for _j in range(P):
            if 0 <= _i + _dr < P and 0 <= _j + _dc < P:
                _PAIRS.append((_i * 3 + _j, (_i + _dr) * 3 + (_j + _dc)))

_WPAD = 58
_FLEN = 56 * _WPAD
_XLEN = 3368


def _fused_kernel(nsteps, xb_ref, xfull_ref,
                  w1x_ref, w1f_ref, g1_ref, b1_ref,
                  w2x_ref, w2f_ref, g2_ref, b2_ref,
                  wm1_ref, gm1_ref, bm1_ref,
                  wm2_ref, gm2_ref, bm2_ref,
                  w3a_ref, w3b_ref, g3_ref, b3_ref,
                  out_ref, nb_ref, f4_ref, pad_ref, col_ref):
    i = pl.program_id(0)

    @pl.when(i < nsteps)
    def _glcm():
        cb = xb_ref.shape[0]
        q = jnp.floor(jnp.clip(xb_ref[...] * (L - 1), 0, L - 1)).astype(
            jnp.bfloat16)
        for k in range(9):
            o = (k // 3) * _WPAD + (k % 3)
            nb_ref[k * cb:(k + 1) * cb, :] = q[:, o:o + _FLEN]
        nbr = [nb_ref[k * cb:(k + 1) * cb, :] for k in range(9)]
        s8 = [n * jnp.bfloat16(L) for n in nbr]
        bins = [s8[a] + nbr[b] for a, b in _PAIRS]
        d = [nbr[a] - nbr[b] for a, b in _PAIRS]

        contrast = None
        for g0 in range(0, 20, 5):
            part = d[g0] * d[g0]
            for dk in d[g0 + 1:g0 + 5]:
                part = part + dk * dk
            pf = part.astype(jnp.float32)
            contrast = pf if contrast is None else contrast + pf

        habs = [jnp.abs(dk).astype(jnp.float32) for dk in d]
        homog = pl.reciprocal(1.0 + habs[0], approx=True)
        for hk in habs[1:]:
            homog = homog + pl.reciprocal(1.0 + hk, approx=True)

        one = jnp.ones_like(bins[0])
        zero = jnp.zeros_like(bins[0])
        inv20 = 1.0 / 20.0
        energy = None
        prod = None
        for k0 in range(0, 20, 4):
            mks = [one, one, one, one]
            for l in range(20):
                bl = bins[l]
                for j in range(4):
                    if l == k0 + j:
                        continue
                    mks[j] = mks[j] + jnp.where(bins[k0 + j] == bl, one,
                                                zero)
            egrp = (mks[0] + mks[1] + mks[2] + mks[3]).astype(jnp.float32)
            energy = egrp if energy is None else energy + egrp
            for j in range(4):
                p = mks[j].astype(jnp.float32) * inv20 + 1e-6
                prod = p if prod is None else prod * p
        logsum = jnp.log(prod)

        fb = jnp.stack([
            jnp.sum(contrast, axis=0),
            jnp.sum(energy, axis=0),
            jnp.sum(logsum, axis=0),
            jnp.sum(homog, axis=0),
        ], axis=0)

        @pl.when(i == 0)
        def _():
            f4_ref[...] = jnp.zeros_like(f4_ref)

        f4_ref[...] += fb

    @pl.when(i == nsteps)
    def _conv():
        cb = xb_ref.shape[0]
        nch = cb * nsteps

        def bnsilu(y, g_ref, b_ref):
            y = y * (g_ref[...] * (1.0 / jnp.sqrt(1.0 + 0.001))) + b_ref[...]
            return y * jax.nn.sigmoid(y)

        def mm(w_ref, a):
            return jax.lax.dot_general(w_ref[...], a,
                                       (((1,), (0,)), ((), ())),
                                       preferred_element_type=jnp.float32)

        x = xfull_ref[:, 59:59 + _FLEN]

        cN = 1.0 / (20.0 * nch)
        rid = jax.lax.broadcasted_iota(jnp.int32, (4, 1), 0)
        scale = jnp.where(rid == 1, 1.0 / (400.0 * nch),
                          jnp.where(rid == 2, -cN, cN))
        f4 = f4_ref[...] * scale

        y1 = mm(w1x_ref, x) + mm(w1f_ref, f4)
        h1 = bnsilu(y1, g1_ref, b1_ref)
        m1 = bnsilu(mm(wm1_ref, h1), gm1_ref, bm1_ref)

        lane = jax.lax.broadcasted_iota(jnp.int32, (1, _FLEN), 1)
        m1 = jnp.where((lane % _WPAD) < 56, m1, 0.0)

        nm = wm2_ref.shape[1] // 9
        pad_ref[...] = jnp.zeros_like(pad_ref)
        pad_ref[:, (_WPAD + 1):(_WPAD + 1) + _FLEN] = m1
        for g in range(9):
            o = (g // 3) * _WPAD + (g % 3)
            col_ref[g * nm:(g + 1) * nm, :] = pad_ref[:, o:o + _FLEN]
        m2 = bnsilu(mm(wm2_ref, col_ref[...]), gm2_ref, bm2_ref)

        mo = h1 + m2
        h2 = bnsilu(mm(w2x_ref, x) + mm(w2f_ref, f4), g2_ref, b2_ref)
        y3 = mm(w3a_ref, mo) + mm(w3b_ref, h2)
        out_ref[...] = bnsilu(y3, g3_ref, b3_ref)


def kernel(x, w_cv1, g_cv1, b_cv1, w_cv2, g_cv2, b_cv2, w_m1, g_m1, b_m1,
           w_m2, g_m2, b_m2, w_cv3, g_cv3, b_cv3):
    B, C, H, W = x.shape
    assert (B, H, W) == (1, 56, 56)
    x0 = x[0]

    xp = jnp.pad(x0, ((0, 0), (1, 1), (1, 1)), mode='edge')
    xp = jnp.pad(xp.reshape(C, _WPAD * _WPAD),
                 ((0, 0), (0, _XLEN - _WPAD * _WPAD)))

    CB = 16
    nsteps = C // CB
    c_ = w_cv1.shape[0]
    ch = w_m1.shape[0]
    c2 = w_cv3.shape[0]
    w1 = w_cv1[:, :, 0, 0]
    w2 = w_cv2[:, :, 0, 0]
    w3 = w_cv3[:, :, 0, 0]
    wm2s = w_m2.transpose(0, 2, 3, 1).reshape(c_, 9 * ch)
    col = lambda v: v[:, None]

    wspecs = [pl.BlockSpec(s, lambda i: (0,) * len(s))
              for s in [(c_, C), (c_, 4), (c_, 1), (c_, 1),
                        (c_, C), (c_, 4), (c_, 1), (c_, 1),
                        (ch, c_), (ch, 1), (ch, 1),
                        (c_, 9 * ch), (c_, 1), (c_, 1),
                        (c2, c_), (c2, c_), (c2, 1), (c2, 1)]]

    out_flat = pl.pallas_call(
        functools.partial(_fused_kernel, nsteps),
        grid=(nsteps + 1,),
        in_specs=[
            pl.BlockSpec((CB, _XLEN),
                         lambda i: (jnp.minimum(i, nsteps - 1), 0)),
            pl.BlockSpec((C, _XLEN), lambda i: (0, 0)),
        ] + wspecs,
        out_specs=pl.BlockSpec((c2, _FLEN), lambda i: (0, 0)),
        out_shape=jax.ShapeDtypeStruct((c2, _FLEN), jnp.float32),
        scratch_shapes=[pltpu.VMEM((9 * CB, _FLEN), jnp.bfloat16),
                        pltpu.VMEM((4, _FLEN), jnp.float32),
                        pltpu.VMEM((ch, _WPAD * _WPAD + 2), jnp.float32),
                        pltpu.VMEM((9 * ch, _FLEN), jnp.float32)],
        compiler_params=pltpu.CompilerParams(
            dimension_semantics=("arbitrary",)),
    )(xp, xp,
      w1[:, :C], w1[:, C:], col(g_cv1), col(b_cv1),
      w2[:, :C], w2[:, C:], col(g_cv2), col(b_cv2),
      w_m1[:, :, 0, 0], col(g_m1), col(b_m1),
      wm2s, col(g_m2), col(b_m2),
      w3[:, :c_], w3[:, c_:], col(g_cv3), col(b_cv3))

    return out_flat.reshape(c2, 56, _WPAD)[None, :, :, :56]


# speedup vs baseline: 1.3622x; 1.3622x over previous
import functools
import jax
import jax.numpy as jnp
from jax.experimental import pallas as pl
from jax.experimental.pallas import tpu as pltpu

P = 3
L = 8
_DIRS = [(0, 1), (1, 0), (1, 1), (1, -1)]
_PAIRS = []
for _dr, _dc in _DIRS:
    for _i in range(P):
        for _j in range(P):
            if 0 <= _i + _dr < P and 0 <= _j + _dc < P:
                _PAIRS.append((_i * 3 + _j, (_i + _dr) * 3 + (_j + _dc)))

_WPAD = 58
_FLEN = 56 * _WPAD
_XLEN = 3368


def _fused_kernel(nsteps, xb_ref, xfull_ref,
                  w1x_ref, w1f_ref, g1_ref, b1_ref,
                  w2x_ref, w2f_ref, g2_ref, b2_ref,
                  wm1_ref, gm1_ref, bm1_ref,
                  wm2_ref, gm2_ref, bm2_ref,
                  w3a_ref, w3b_ref, g3_ref, b3_ref,
                  out_ref, nb_ref, f4_ref, pad_ref, col_ref):
    i = pl.program_id(0)

    @pl.when(i < nsteps)
    def _glcm():
        cb = xb_ref.shape[0]
        q = jnp.floor(jnp.clip(xb_ref[...] * (L - 1), 0, L - 1)).astype(
            jnp.bfloat16)
        for k in range(9):
            o = (k // 3) * _WPAD + (k % 3)
            nb_ref[k * cb:(k + 1) * cb, :] = q[:, o:o + _FLEN]

        @pl.when(i == 0)
        def _():
            f4_ref[...] = jnp.zeros_like(f4_ref)

        for c0 in range(0, _FLEN, 512):
            cl = min(512, _FLEN - c0)
            nbr = [nb_ref[k * cb:(k + 1) * cb, c0:c0 + cl] for k in range(9)]
            s8 = [n * jnp.bfloat16(L) for n in nbr]
            bins = [s8[a] + nbr[b] for a, b in _PAIRS]
            d = [nbr[a] - nbr[b] for a, b in _PAIRS]

            contrast = None
            for g0 in range(0, 20, 5):
                part = d[g0] * d[g0]
                for dk in d[g0 + 1:g0 + 5]:
                    part = part + dk * dk
                pf = part.astype(jnp.float32)
                contrast = pf if contrast is None else contrast + pf

            habs = [jnp.abs(dk).astype(jnp.float32) for dk in d]
            homog = pl.reciprocal(1.0 + habs[0], approx=True)
            for hk in habs[1:]:
                homog = homog + pl.reciprocal(1.0 + hk, approx=True)

            one = jnp.ones_like(bins[0])
            zero = jnp.zeros_like(bins[0])
            inv20 = 1.0 / 20.0
            m = [one] * 20
            for k in range(20):
                bk = bins[k]
                for l in range(k + 1, 20):
                    e = jnp.where(bk == bins[l], one, zero)
                    m[k] = m[k] + e
                    m[l] = m[l] + e
            energy = None
            prod = None
            for k0 in range(0, 20, 4):
                egrp = (m[k0] + m[k0 + 1] + m[k0 + 2]
                        + m[k0 + 3]).astype(jnp.float32)
                energy = egrp if energy is None else energy + egrp
                for j in range(4):
                    p = m[k0 + j].astype(jnp.float32) * inv20 + 1e-6
                    prod = p if prod is None else prod * p
            logsum = jnp.log(prod)

            fb = jnp.stack([
                jnp.sum(contrast, axis=0),
                jnp.sum(energy, axis=0),
                jnp.sum(logsum, axis=0),
                jnp.sum(homog, axis=0),
            ], axis=0)
            f4_ref[:, c0:c0 + cl] += fb

    @pl.when(i == nsteps)
    def _conv():
        cb = xb_ref.shape[0]
        nch = cb * nsteps

        def bnsilu(y, g_ref, b_ref):
            y = y * (g_ref[...] * (1.0 / jnp.sqrt(1.0 + 0.001))) + b_ref[...]
            return y * jax.nn.sigmoid(y)

        def mm(w_ref, a):
            return jax.lax.dot_general(w_ref[...], a,
                                       (((1,), (0,)), ((), ())),
                                       preferred_element_type=jnp.float32)

        x = xfull_ref[:, 59:59 + _FLEN]

        cN = 1.0 / (20.0 * nch)
        rid = jax.lax.broadcasted_iota(jnp.int32, (4, 1), 0)
        scale = jnp.where(rid == 1, 1.0 / (400.0 * nch),
                          jnp.where(rid == 2, -cN, cN))
        f4 = f4_ref[...] * scale

        y1 = mm(w1x_ref, x) + mm(w1f_ref, f4)
        h1 = bnsilu(y1, g1_ref, b1_ref)
        m1 = bnsilu(mm(wm1_ref, h1), gm1_ref, bm1_ref)

        lane = jax.lax.broadcasted_iota(jnp.int32, (1, _FLEN), 1)
        m1 = jnp.where((lane % _WPAD) < 56, m1, 0.0)

        nm = wm2_ref.shape[1] // 9
        pad_ref[...] = jnp.zeros_like(pad_ref)
        pad_ref[:, (_WPAD + 1):(_WPAD + 1) + _FLEN] = m1
        for g in range(9):
            o = (g // 3) * _WPAD + (g % 3)
            col_ref[g * nm:(g + 1) * nm, :] = pad_ref[:, o:o + _FLEN]
        m2 = bnsilu(mm(wm2_ref, col_ref[...]), gm2_ref, bm2_ref)

        mo = h1 + m2
        h2 = bnsilu(mm(w2x_ref, x) + mm(w2f_ref, f4), g2_ref, b2_ref)
        y3 = mm(w3a_ref, mo) + mm(w3b_ref, h2)
        out_ref[...] = bnsilu(y3, g3_ref, b3_ref)


def kernel(x, w_cv1, g_cv1, b_cv1, w_cv2, g_cv2, b_cv2, w_m1, g_m1, b_m1,
           w_m2, g_m2, b_m2, w_cv3, g_cv3, b_cv3):
    B, C, H, W = x.shape
    assert (B, H, W) == (1, 56, 56)
    x0 = x[0]

    xp = jnp.pad(x0, ((0, 0), (1, 1), (1, 1)), mode='edge')
    xp = jnp.pad(xp.reshape(C, _WPAD * _WPAD),
                 ((0, 0), (0, _XLEN - _WPAD * _WPAD)))

    CB = 8
    nsteps = C // CB
    c_ = w_cv1.shape[0]
    ch = w_m1.shape[0]
    c2 = w_cv3.shape[0]
    w1 = w_cv1[:, :, 0, 0]
    w2 = w_cv2[:, :, 0, 0]
    w3 = w_cv3[:, :, 0, 0]
    wm2s = w_m2.transpose(0, 2, 3, 1).reshape(c_, 9 * ch)
    col = lambda v: v[:, None]

    wspecs = [pl.BlockSpec(s, lambda i: (0,) * len(s))
              for s in [(c_, C), (c_, 4), (c_, 1), (c_, 1),
                        (c_, C), (c_, 4), (c_, 1), (c_, 1),
                        (ch, c_), (ch, 1), (ch, 1),
                        (c_, 9 * ch), (c_, 1), (c_, 1),
                        (c2, c_), (c2, c_), (c2, 1), (c2, 1)]]

    out_flat = pl.pallas_call(
        functools.partial(_fused_kernel, nsteps),
        grid=(nsteps + 1,),
        in_specs=[
            pl.BlockSpec((CB, _XLEN),
                         lambda i: (jnp.minimum(i, nsteps - 1), 0)),
            pl.BlockSpec((C, _XLEN), lambda i: (0, 0)),
        ] + wspecs,
        out_specs=pl.BlockSpec((c2, _FLEN), lambda i: (0, 0)),
        out_shape=jax.ShapeDtypeStruct((c2, _FLEN), jnp.float32),
        scratch_shapes=[pltpu.VMEM((9 * CB, _FLEN), jnp.bfloat16),
                        pltpu.VMEM((4, _FLEN), jnp.float32),
                        pltpu.VMEM((ch, _WPAD * _WPAD + 2), jnp.float32),
                        pltpu.VMEM((9 * ch, _FLEN), jnp.float32)],
        compiler_params=pltpu.CompilerParams(
            dimension_semantics=("arbitrary",)),
    )(xp, xp,
      w1[:, :C], w1[:, C:], col(g_cv1), col(b_cv1),
      w2[:, :C], w2[:, C:], col(g_cv2), col(b_cv2),
      w_m1[:, :, 0, 0], col(g_m1), col(b_m1),
      wm2s, col(g_m2), col(b_m2),
      w3[:, :c_], w3[:, c_:], col(g_cv3), col(b_cv3))

    return out_flat.reshape(c2, 56, _WPAD)[None, :, :, :56]
